# Initial kernel scaffold; baseline (speedup 1.0000x reference)
import jax
import jax.numpy as jnp
from jax.experimental import pallas as pl


def kernel(X, table):
    raise NotImplementedError("write your pallas kernel here")


# speedup vs baseline: 1.2032x; 1.2032x over previous
import functools

import jax
import jax.numpy as jnp
from jax import lax
from jax.experimental import pallas as pl
from jax.experimental.pallas import tpu as pltpu
from jax.experimental.pallas import tpu_sc as plsc

_NUM_CORES = 2
_NUM_SUBCORES = 16
_NW = _NUM_CORES * _NUM_SUBCORES

_IDX_W = 104
_G_PER_CHUNK = 8
_CHUNK = _IDX_W * _G_PER_CHUNK
_N_CHUNK = 4


def _make_sc_gather(D, B):
  assert B == _NW * _N_CHUNK * _CHUNK
  rows_per_w = _N_CHUNK * _CHUNK
  idx_rows_per_w = _N_CHUNK * _G_PER_CHUNK

  mesh = plsc.VectorSubcoreMesh(core_axis_name="c", subcore_axis_name="s")

  @functools.partial(
      pl.kernel,
      mesh=mesh,
      compiler_params=pltpu.CompilerParams(use_tc_tiling_on_sc=False),
      out_type=jax.ShapeDtypeStruct((B, D), jnp.float32),
      scratch_types=[
          pltpu.VMEM((idx_rows_per_w, _IDX_W), jnp.int32),
          pltpu.VMEM((2, _CHUNK, D), jnp.float32),
          pltpu.SemaphoreType.DMA,
      ],
  )
  def gather_kernel(idx_hbm, table_hbm, out_hbm, idx_v, rows_v, gsem):
    wid = lax.axis_index("s") * _NUM_CORES + lax.axis_index("c")
    base = wid * rows_per_w

    pltpu.sync_copy(idx_hbm.at[pl.ds(wid * idx_rows_per_w, idx_rows_per_w)],
                    idx_v)

    @pl.loop(0, _N_CHUNK)
    def chunk_loop(c):
      b = lax.rem(c, 2)
      copies = []
      for j in range(_G_PER_CHUNK):
        copies.append(
            pltpu.async_copy(
                table_hbm.at[idx_v.at[c * _G_PER_CHUNK + j]],
                rows_v.at[b, pl.ds(j * _IDX_W, _IDX_W)],
                gsem,
            ))

      @pl.when(c > 0)
      def _():
        pltpu.sync_copy(rows_v.at[1 - b],
                        out_hbm.at[pl.ds(base + (c - 1) * _CHUNK, _CHUNK)])

      for cp in copies:
        cp.wait()

    pltpu.sync_copy(
        rows_v.at[(_N_CHUNK - 1) % 2],
        out_hbm.at[pl.ds(base + (_N_CHUNK - 1) * _CHUNK, _CHUNK)])

  return gather_kernel


def kernel(X, table):
  Bt, F = X.shape
  V, D = table.shape
  B = Bt * F
  idx = X.reshape(B // _IDX_W, _IDX_W).astype(jnp.int32)
  out = _make_sc_gather(D, B)(idx, table)
  return out.reshape(Bt, F, D)
